# Initial kernel scaffold; baseline (speedup 1.0000x reference)
import jax
import jax.numpy as jnp
from jax.experimental import pallas as pl


def kernel(iid, edge_index, edge_t, segment_ids, last_nodes, emb, Wxr, bxr, Wxz, bxz, Wxh, bxh, Whr, bhr, Whz, bhz, Whh, bhh, Wu, Wv, bv, We, Wsr):
    raise NotImplementedError("write your pallas kernel here")


# speedup vs baseline: 5.5061x; 5.5061x over previous
import jax
import jax.numpy as jnp
from jax import lax
from jax.experimental import pallas as pl
from jax.experimental.pallas import tpu as pltpu
from jax.experimental.pallas import tpu_sc as plsc

N = 10000
E = 320000
D = 128
V = 100000
B = 100
SEG = N // B
SCALE = 12.0

NC = 2
NS = 16
NW = NC * NS
NPAD = 10240
DUMP = N
EPT = E // NW
CH = 128
NCHUNK = EPT // CH
TAIL = EPT - NCHUNK * CH
GPT = 320
GCH = 64
NT = 10
RT = N // NT
VT = 2048
NVT = (V + VT - 1) // VT

_mesh = plsc.VectorSubcoreMesh(
    core_axis_name="c", subcore_axis_name="s", num_cores=NC, num_subcores=NS)


def _k1_body(iid_hbm, src_hbm, dst_hbm, et_hbm, emb_hbm, z1_hbm,
             g_out, degO_out, degI_out, de_out, tmax_out,
             idx_v, rows_v, src_v, dst_v, et_v, m_v, de_v,
             src_t, dst_t, et_t, m_t, de_t, tm_v,
             degO_acc, degI_acc, sem):
    c = lax.axis_index("c")
    s = lax.axis_index("s")
    w = c * NS + s

    @pl.when(s == 0)
    def _():
        pltpu.sync_copy(z1_hbm, degO_acc)
        pltpu.sync_copy(z1_hbm, degI_acc)

    start = jnp.minimum(w * GPT, N - GPT)
    for j in range(GPT // GCH):
        pltpu.sync_copy(iid_hbm.at[pl.ds(start + j * GCH, GCH)], idx_v)
        pltpu.async_copy(emb_hbm.at[idx_v], rows_v, sem).wait()
        pltpu.sync_copy(rows_v, g_out.at[pl.ds(start + j * GCH, GCH)])

    plsc.subcore_barrier()

    tm_v[...] = jnp.full((16,), -3.0e38, jnp.float32)
    base = w * EPT

    def chunk(i, _):
        off = base + i * CH
        pltpu.sync_copy(src_hbm.at[pl.ds(off, CH)], src_v)
        pltpu.sync_copy(dst_hbm.at[pl.ds(off, CH)], dst_v)
        pltpu.sync_copy(et_hbm.at[pl.ds(off, CH)], et_v)
        for j in range(CH // 16):
            t = et_v[pl.ds(j * 16, 16)]
            d = dst_v[pl.ds(j * 16, 16)]
            msk = t <= 0.0
            m_v[pl.ds(j * 16, 16)] = jnp.where(msk, 1.0, 0.0)
            de_v[pl.ds(j * 16, 16)] = jnp.where(msk, d, DUMP)
            tm_v[...] = jnp.maximum(tm_v[...], t)
        pltpu.sync_copy(m_v, degO_acc.at[src_v], add=True)
        pltpu.sync_copy(m_v, degI_acc.at[dst_v], add=True)
        pltpu.sync_copy(de_v, de_out.at[pl.ds(off, CH)])
        return 0

    lax.fori_loop(0, NCHUNK, chunk, 0)

    off = base + NCHUNK * CH
    pltpu.sync_copy(src_hbm.at[pl.ds(off, TAIL)], src_t)
    pltpu.sync_copy(dst_hbm.at[pl.ds(off, TAIL)], dst_t)
    pltpu.sync_copy(et_hbm.at[pl.ds(off, TAIL)], et_t)
    t = et_t[...]
    msk = t <= 0.0
    m_t[...] = jnp.where(msk, 1.0, 0.0)
    de_t[...] = jnp.where(msk, dst_t[...], DUMP)
    tm_v[...] = jnp.maximum(tm_v[...], t)
    pltpu.sync_copy(m_t, degO_acc.at[src_t], add=True)
    pltpu.sync_copy(m_t, degI_acc.at[dst_t], add=True)
    pltpu.sync_copy(de_t, de_out.at[pl.ds(off, TAIL)])

    pltpu.sync_copy(tm_v, tmax_out.at[w])

    plsc.subcore_barrier()
    pltpu.sync_copy(degO_acc.at[pl.ds(s * 640, 640)],
                    degO_out.at[c, pl.ds(s * 640, 640)])
    pltpu.sync_copy(degI_acc.at[pl.ds(s * 640, 640)],
                    degI_out.at[c, pl.ds(s * 640, 640)])


_k1 = pl.kernel(
    _k1_body,
    out_type=(
        jax.ShapeDtypeStruct((N, D), jnp.float32),
        jax.ShapeDtypeStruct((NC, NPAD), jnp.float32),
        jax.ShapeDtypeStruct((NC, NPAD), jnp.float32),
        jax.ShapeDtypeStruct((E,), jnp.int32),
        jax.ShapeDtypeStruct((NW, 16), jnp.float32),
    ),
    mesh=_mesh,
    scratch_types=[
        pltpu.VMEM((GCH,), jnp.int32),
        pltpu.VMEM((GCH, D), jnp.float32),
        pltpu.VMEM((CH,), jnp.int32),
        pltpu.VMEM((CH,), jnp.int32),
        pltpu.VMEM((CH,), jnp.float32),
        pltpu.VMEM((CH,), jnp.float32),
        pltpu.VMEM((CH,), jnp.int32),
        pltpu.VMEM((TAIL,), jnp.int32),
        pltpu.VMEM((TAIL,), jnp.int32),
        pltpu.VMEM((TAIL,), jnp.float32),
        pltpu.VMEM((TAIL,), jnp.float32),
        pltpu.VMEM((TAIL,), jnp.int32),
        pltpu.VMEM((16,), jnp.float32),
        pltpu.VMEM_SHARED((NPAD,), jnp.float32),
        pltpu.VMEM_SHARED((NPAD,), jnp.float32),
        pltpu.SemaphoreType.DMA,
    ],
)


def _agg_body(x_hbm, src_hbm, de_hbm, z2_hbm, p_out,
              src_v, de_v, rows_v, src_t, de_t, rows_t, acc, sem):
    c = lax.axis_index("c")
    s = lax.axis_index("s")
    w = c * NS + s

    pltpu.sync_copy(z2_hbm.at[pl.ds(s * 640, 640)], acc.at[pl.ds(s * 640, 640)])
    plsc.subcore_barrier()

    base = w * EPT

    def chunk(i, _):
        off = base + i * CH
        pltpu.sync_copy(src_hbm.at[pl.ds(off, CH)], src_v)
        pltpu.sync_copy(de_hbm.at[pl.ds(off, CH)], de_v)
        pltpu.async_copy(x_hbm.at[src_v], rows_v, sem).wait()
        pltpu.sync_copy(rows_v, acc.at[de_v], add=True)
        return 0

    lax.fori_loop(0, NCHUNK, chunk, 0)

    off = base + NCHUNK * CH
    pltpu.sync_copy(src_hbm.at[pl.ds(off, TAIL)], src_t)
    pltpu.sync_copy(de_hbm.at[pl.ds(off, TAIL)], de_t)
    pltpu.async_copy(x_hbm.at[src_t], rows_t, sem).wait()
    pltpu.sync_copy(rows_t, acc.at[de_t], add=True)

    plsc.subcore_barrier()
    pltpu.sync_copy(acc.at[pl.ds(s * 640, 640)],
                    p_out.at[c, pl.ds(s * 640, 640)])


_agg = pl.kernel(
    _agg_body,
    out_type=jax.ShapeDtypeStruct((NC, NPAD, D), jnp.float32),
    mesh=_mesh,
    scratch_types=[
        pltpu.VMEM((CH,), jnp.int32),
        pltpu.VMEM((CH,), jnp.int32),
        pltpu.VMEM((CH, D), jnp.float32),
        pltpu.VMEM((TAIL,), jnp.int32),
        pltpu.VMEM((TAIL,), jnp.int32),
        pltpu.VMEM((TAIL, D), jnp.float32),
        pltpu.VMEM_SHARED((NPAD, D), jnp.float32),
        pltpu.SemaphoreType.DMA,
    ],
)


def _k2_body(g_ref, degO_ref, f_out, fs_out):
    g = g_ref[...]
    nrm = jnp.sqrt(jnp.sum(g * g, axis=1, keepdims=True))
    f = g / (nrm + 1e-12)
    dd = degO_ref[0] + degO_ref[1]
    ns = lax.rsqrt(jnp.maximum(dd, 1.0))
    f_out[...] = f
    fs_out[...] = f * ns


def _k2(g, degO3):
    return pl.pallas_call(
        _k2_body,
        grid=(NT,),
        in_specs=[
            pl.BlockSpec((RT, D), lambda i: (i, 0)),
            pl.BlockSpec((NC, RT, 1), lambda i: (0, i, 0)),
        ],
        out_specs=[
            pl.BlockSpec((RT, D), lambda i: (i, 0)),
            pl.BlockSpec((RT, D), lambda i: (i, 0)),
        ],
        out_shape=[
            jax.ShapeDtypeStruct((N, D), jnp.float32),
            jax.ShapeDtypeStruct((N, D), jnp.float32),
        ],
    )(g, degO3)


def _k4_body(p_ref, degI_ref, degO_ref, f_ref,
             wxr_ref, bxr_ref, wxz_ref, bxz_ref, wxh_ref, bxh_ref,
             whr_ref, bhr_ref, whz_ref, bhz_ref,
             rhs_out, z_out, xh_out):
    nd = lax.rsqrt(jnp.maximum(degI_ref[0] + degI_ref[1], 1.0))
    agg = (p_ref[0] + p_ref[1]) * nd
    wr = wxr_ref[...] + whr_ref[...]
    wz = wxz_ref[...] + whz_ref[...]
    br = bxr_ref[...] + bhr_ref[...]
    bz = bxz_ref[...] + bhz_ref[...]
    r = jax.nn.sigmoid(jnp.dot(agg, wr, preferred_element_type=jnp.float32) + br)
    z = jax.nn.sigmoid(jnp.dot(agg, wz, preferred_element_type=jnp.float32) + bz)
    xh = jnp.dot(agg, wxh_ref[...], preferred_element_type=jnp.float32) + bxh_ref[...]
    ns = lax.rsqrt(jnp.maximum(degO_ref[0] + degO_ref[1], 1.0))
    rhs_out[...] = r * f_ref[...] * ns
    z_out[...] = z
    xh_out[...] = xh


def _k4(p, degI3, degO3, feat, wxr, bxr, wxz, bxz, wxh, bxh, whr, bhr, whz, bhz):
    wspec = pl.BlockSpec((D, D), lambda i: (0, 0))
    bspec = pl.BlockSpec((1, D), lambda i: (0, 0))
    dspec = pl.BlockSpec((NC, RT, 1), lambda i: (0, i, 0))
    nspec = pl.BlockSpec((RT, D), lambda i: (i, 0))
    return pl.pallas_call(
        _k4_body,
        grid=(NT,),
        in_specs=[
            pl.BlockSpec((NC, RT, D), lambda i: (0, i, 0)),
            dspec, dspec, nspec,
            wspec, bspec, wspec, bspec, wspec, bspec,
            wspec, bspec, wspec, bspec,
        ],
        out_specs=[nspec, nspec, nspec],
        out_shape=[
            jax.ShapeDtypeStruct((N, D), jnp.float32),
            jax.ShapeDtypeStruct((N, D), jnp.float32),
            jax.ShapeDtypeStruct((N, D), jnp.float32),
        ],
    )(p, degI3, degO3, feat, wxr, bxr, wxz, bxz, wxh, bxh, whr, bhr, whz, bhz)


def _k6_body(q_ref, degI_ref, z_ref, xh_ref, f_ref, whh_ref, bhh_ref,
             tmax_ref, wu_ref, f2n_out, fu_out):
    nd = lax.rsqrt(jnp.maximum(degI_ref[0] + degI_ref[1], 1.0))
    agg2 = (q_ref[0] + q_ref[1]) * nd
    u = jnp.tanh(xh_ref[...]
                 + jnp.dot(agg2, whh_ref[...], preferred_element_type=jnp.float32)
                 + bhh_ref[...])
    f = f_ref[...]
    dh = (1.0 - z_ref[...]) * (u - f)
    dt = jnp.max(tmax_ref[...]) * 0.1
    f2 = f + dt * dh
    f2n = f2 / jnp.sqrt(jnp.sum(f2 * f2, axis=1, keepdims=True))
    f2n_out[...] = f2n
    fu_out[...] = jnp.dot(f2n, wu_ref[...], preferred_element_type=jnp.float32)


def _k6(q, degI3, z, xh, feat, whh, bhh, tmax, wu):
    nspec = pl.BlockSpec((RT, D), lambda i: (i, 0))
    return pl.pallas_call(
        _k6_body,
        grid=(NT,),
        in_specs=[
            pl.BlockSpec((NC, RT, D), lambda i: (0, i, 0)),
            pl.BlockSpec((NC, RT, 1), lambda i: (0, i, 0)),
            nspec, nspec, nspec,
            pl.BlockSpec((D, D), lambda i: (0, 0)),
            pl.BlockSpec((1, D), lambda i: (0, 0)),
            pl.BlockSpec((NW, 16), lambda i: (0, 0)),
            pl.BlockSpec((D, D), lambda i: (0, 0)),
        ],
        out_specs=[nspec, nspec],
        out_shape=[
            jax.ShapeDtypeStruct((N, D), jnp.float32),
            jax.ShapeDtypeStruct((N, D), jnp.float32),
        ],
    )(q, degI3, z, xh, feat, whh, bhh, tmax, wu)


def _attn_body(f3_ref, fu3_ref, wv_ref, bv_ref, we_ref, wsr_ref, sr_out):
    f3 = f3_ref[...]
    fu3 = fu3_ref[...]
    last = f3[:, SEG - 1, :]
    fv = jnp.dot(last, wv_ref[...], preferred_element_type=jnp.float32) \
        + bv_ref[...]
    sig = jax.nn.sigmoid(fu3 + fv[:, None, :])
    e3 = jnp.sum(sig * we_ref[...], axis=2, keepdims=True)
    emax = jnp.max(e3, axis=1, keepdims=True)
    ee = jnp.exp(e3 - emax)
    es = jnp.sum(ee, axis=1, keepdims=True)
    alpha = ee / es
    sr_g = jnp.sum(f3 * alpha, axis=1)
    srh = jnp.concatenate([last, sr_g], axis=1)
    sr0 = jnp.dot(srh, wsr_ref[...], preferred_element_type=jnp.float32)
    nrm = jnp.sqrt(jnp.sum(sr0 * sr0, axis=1, keepdims=True))
    sr_out[...] = sr0 / (nrm + 1e-12)


def _attn(f3, fu3, wv, bv, we, wsr):
    return pl.pallas_call(
        _attn_body,
        out_shape=jax.ShapeDtypeStruct((B, D), jnp.float32),
    )(f3, fu3, wv, bv, we, wsr)


def _k7_body(sr_ref, emb_ref, out_ref):
    et = emb_ref[...]
    sr = sr_ref[...]
    ones = jnp.ones((1, D), jnp.float32)
    rn2 = lax.dot_general(ones, et * et, (((1,), (1,)), ((), ())),
                          preferred_element_type=jnp.float32)
    logits = lax.dot_general(sr, et, (((1,), (1,)), ((), ())),
                             preferred_element_type=jnp.float32)
    out_ref[...] = logits * (SCALE / (jnp.sqrt(rn2) + 1e-12))


def _k7(sr, emb):
    return pl.pallas_call(
        _k7_body,
        grid=(NVT,),
        in_specs=[
            pl.BlockSpec((B, D), lambda i: (0, 0)),
            pl.BlockSpec((VT, D), lambda i: (i, 0)),
        ],
        out_specs=pl.BlockSpec((B, VT), lambda i: (0, i)),
        out_shape=jax.ShapeDtypeStruct((B, V), jnp.float32),
    )(sr, emb)


def kernel(iid, edge_index, edge_t, segment_ids, last_nodes, emb,
           Wxr, bxr, Wxz, bxz, Wxh, bxh, Whr, bhr, Whz, bhz, Whh, bhh,
           Wu, Wv, bv, We, Wsr):
    iid = iid.astype(jnp.int32)
    src = edge_index[0].astype(jnp.int32)
    dst = edge_index[1].astype(jnp.int32)
    edge_t = edge_t.astype(jnp.float32)
    z1 = jnp.zeros((NPAD,), jnp.float32)
    z2 = jnp.zeros((NPAD, D), jnp.float32)

    g, degO, degI, de, tmax = _k1(iid, src, dst, edge_t, emb, z1)
    degO3 = degO.reshape(NC, NPAD, 1)
    degI3 = degI.reshape(NC, NPAD, 1)

    feat, fs = _k2(g, degO3)
    p = _agg(fs, src, de, z2)
    rhs, z, xh = _k4(p, degI3, degO3, feat,
                     Wxr, bxr.reshape(1, D), Wxz, bxz.reshape(1, D),
                     Wxh, bxh.reshape(1, D), Whr, bhr.reshape(1, D),
                     Whz, bhz.reshape(1, D))
    q = _agg(rhs, src, de, z2)
    f2n, fu = _k6(q, degI3, z, xh, feat, Whh, bhh.reshape(1, D), tmax, Wu)

    f3 = f2n.reshape(B, SEG, D)
    fu3 = fu.reshape(B, SEG, D)
    sr = _attn(f3, fu3, Wv, bv.reshape(1, D), We.reshape(1, 1, D), Wsr)

    return _k7(sr, emb)


# speedup vs baseline: 8.5199x; 1.5473x over previous
import jax
import jax.numpy as jnp
from jax import lax
from jax.experimental import pallas as pl
from jax.experimental.pallas import tpu as pltpu
from jax.experimental.pallas import tpu_sc as plsc

N = 10000
E = 320000
D = 128
V = 100000
B = 100
SEG = N // B
SCALE = 12.0

NC = 2
NS = 16
NW = NC * NS
NPAD = 10240
DUMP = N
EPT = E // NW
CH = 128
NCHUNK = EPT // CH
TAIL = EPT - NCHUNK * CH
GPT = 320
GCH = 64
NT = 10
RT = N // NT
VT = 2048
NVT = (V + VT - 1) // VT

_mesh = plsc.VectorSubcoreMesh(
    core_axis_name="c", subcore_axis_name="s", num_cores=NC, num_subcores=NS)


def _k1_body(iid_hbm, src_hbm, dst_hbm, et_hbm, emb_hbm, z1_hbm,
             g_out, degO_out, degI_out, de_out, tmax_out,
             idx_v, rows_v,
             src_v0, dst_v0, et_v0, src_v1, dst_v1, et_v1, m_v, de_v,
             src_t, dst_t, et_t, m_t, de_t, tm_v,
             degO_acc, degI_acc, sem, isem0, isem1):
    c = lax.axis_index("c")
    s = lax.axis_index("s")
    w = c * NS + s

    @pl.when(s == 0)
    def _():
        pltpu.sync_copy(z1_hbm, degO_acc)
        pltpu.sync_copy(z1_hbm, degI_acc)

    start = jnp.minimum(w * GPT, N - GPT)
    pltpu.sync_copy(iid_hbm.at[pl.ds(start, GPT)], idx_v)
    for j in range(GPT // GCH):
        pltpu.async_copy(emb_hbm.at[idx_v.at[pl.ds(j * GCH, GCH)]],
                         rows_v.at[pl.ds(j * GCH, GCH)], sem)

    plsc.subcore_barrier()

    tm_v[...] = jnp.full((16,), -3.0e38, jnp.float32)
    base = w * EPT

    def _issue(i, sv, dv, ev, isem):
        off = base + i * CH
        pltpu.async_copy(src_hbm.at[pl.ds(off, CH)], sv, isem)
        pltpu.async_copy(dst_hbm.at[pl.ds(off, CH)], dv, isem)
        pltpu.async_copy(et_hbm.at[pl.ds(off, CH)], ev, isem)

    def _wait(sv, dv, ev, isem):
        pltpu.make_async_copy(src_hbm.at[pl.ds(base, CH)], sv, isem).wait()
        pltpu.make_async_copy(dst_hbm.at[pl.ds(base, CH)], dv, isem).wait()
        pltpu.make_async_copy(et_hbm.at[pl.ds(base, CH)], ev, isem).wait()

    def _process(i, sv, dv, ev):
        off = base + i * CH
        for j in range(CH // 16):
            t = ev[pl.ds(j * 16, 16)]
            d = dv[pl.ds(j * 16, 16)]
            msk = t <= 0.0
            m_v[pl.ds(j * 16, 16)] = jnp.where(msk, 1.0, 0.0)
            de_v[pl.ds(j * 16, 16)] = jnp.where(msk, d, DUMP)
            tm_v[...] = jnp.maximum(tm_v[...], t)
        pltpu.sync_copy(m_v, degO_acc.at[sv], add=True)
        pltpu.sync_copy(m_v, degI_acc.at[dv], add=True)
        pltpu.sync_copy(de_v, de_out.at[pl.ds(off, CH)])

    NG = NCHUNK // 2
    _issue(0, src_v0, dst_v0, et_v0, isem0)

    def group(g, _):
        i0 = 2 * g
        _issue(i0 + 1, src_v1, dst_v1, et_v1, isem1)
        _wait(src_v0, dst_v0, et_v0, isem0)
        _process(i0, src_v0, dst_v0, et_v0)

        @pl.when(g < NG - 1)
        def _():
            _issue(i0 + 2, src_v0, dst_v0, et_v0, isem0)

        _wait(src_v1, dst_v1, et_v1, isem1)
        _process(i0 + 1, src_v1, dst_v1, et_v1)
        return 0

    lax.fori_loop(0, NG, group, 0)

    off = base + NCHUNK * CH
    pltpu.sync_copy(src_hbm.at[pl.ds(off, TAIL)], src_t)
    pltpu.sync_copy(dst_hbm.at[pl.ds(off, TAIL)], dst_t)
    pltpu.sync_copy(et_hbm.at[pl.ds(off, TAIL)], et_t)
    t = et_t[...]
    msk = t <= 0.0
    m_t[...] = jnp.where(msk, 1.0, 0.0)
    de_t[...] = jnp.where(msk, dst_t[...], DUMP)
    tm_v[...] = jnp.maximum(tm_v[...], t)
    pltpu.sync_copy(m_t, degO_acc.at[src_t], add=True)
    pltpu.sync_copy(m_t, degI_acc.at[dst_t], add=True)
    pltpu.sync_copy(de_t, de_out.at[pl.ds(off, TAIL)])

    pltpu.sync_copy(tm_v, tmax_out.at[w])

    for j in range(GPT // GCH):
        pltpu.make_async_copy(emb_hbm.at[idx_v.at[pl.ds(j * GCH, GCH)]],
                              rows_v.at[pl.ds(j * GCH, GCH)], sem).wait()
    pltpu.sync_copy(rows_v, g_out.at[pl.ds(start, GPT)])

    plsc.subcore_barrier()
    pltpu.sync_copy(degO_acc.at[pl.ds(s * 640, 640)],
                    degO_out.at[c, pl.ds(s * 640, 640)])
    pltpu.sync_copy(degI_acc.at[pl.ds(s * 640, 640)],
                    degI_out.at[c, pl.ds(s * 640, 640)])


_k1 = pl.kernel(
    _k1_body,
    out_type=(
        jax.ShapeDtypeStruct((N, D), jnp.float32),
        jax.ShapeDtypeStruct((NC, NPAD), jnp.float32),
        jax.ShapeDtypeStruct((NC, NPAD), jnp.float32),
        jax.ShapeDtypeStruct((E,), jnp.int32),
        jax.ShapeDtypeStruct((NW, 16), jnp.float32),
    ),
    mesh=_mesh,
    scratch_types=[
        pltpu.VMEM((GPT,), jnp.int32),
        pltpu.VMEM((GPT, D), jnp.float32),
        pltpu.VMEM((CH,), jnp.int32),
        pltpu.VMEM((CH,), jnp.int32),
        pltpu.VMEM((CH,), jnp.float32),
        pltpu.VMEM((CH,), jnp.int32),
        pltpu.VMEM((CH,), jnp.int32),
        pltpu.VMEM((CH,), jnp.float32),
        pltpu.VMEM((CH,), jnp.float32),
        pltpu.VMEM((CH,), jnp.int32),
        pltpu.VMEM((TAIL,), jnp.int32),
        pltpu.VMEM((TAIL,), jnp.int32),
        pltpu.VMEM((TAIL,), jnp.float32),
        pltpu.VMEM((TAIL,), jnp.float32),
        pltpu.VMEM((TAIL,), jnp.int32),
        pltpu.VMEM((16,), jnp.float32),
        pltpu.VMEM_SHARED((NPAD,), jnp.float32),
        pltpu.VMEM_SHARED((NPAD,), jnp.float32),
        pltpu.SemaphoreType.DMA,
        pltpu.SemaphoreType.DMA,
        pltpu.SemaphoreType.DMA,
    ],
)


def _agg_body(x_hbm, src_hbm, de_hbm, z2_hbm, p_out,
              src_v0, de_v0, rows_v0, src_v1, de_v1, rows_v1,
              src_t, de_t, rows_t, acc, gsem0, gsem1):
    c = lax.axis_index("c")
    s = lax.axis_index("s")
    w = c * NS + s

    pltpu.sync_copy(z2_hbm.at[pl.ds(s * 640, 640)], acc.at[pl.ds(s * 640, 640)])
    plsc.subcore_barrier()

    base = w * EPT

    def _issue(i, sv, dv, rv, gsem):
        off = base + i * CH
        pltpu.sync_copy(src_hbm.at[pl.ds(off, CH)], sv)
        pltpu.sync_copy(de_hbm.at[pl.ds(off, CH)], dv)
        pltpu.async_copy(x_hbm.at[sv], rv, gsem)

    NG = NCHUNK // 2
    _issue(0, src_v0, de_v0, rows_v0, gsem0)

    def group(g, _):
        i0 = 2 * g
        _issue(i0 + 1, src_v1, de_v1, rows_v1, gsem1)
        pltpu.make_async_copy(x_hbm.at[src_v0], rows_v0, gsem0).wait()
        pltpu.sync_copy(rows_v0, acc.at[de_v0], add=True)

        @pl.when(g < NG - 1)
        def _():
            _issue(i0 + 2, src_v0, de_v0, rows_v0, gsem0)

        pltpu.make_async_copy(x_hbm.at[src_v1], rows_v1, gsem1).wait()
        pltpu.sync_copy(rows_v1, acc.at[de_v1], add=True)
        return 0

    lax.fori_loop(0, NG, group, 0)

    off = base + NCHUNK * CH
    pltpu.sync_copy(src_hbm.at[pl.ds(off, TAIL)], src_t)
    pltpu.sync_copy(de_hbm.at[pl.ds(off, TAIL)], de_t)
    pltpu.async_copy(x_hbm.at[src_t], rows_t, gsem0).wait()
    pltpu.sync_copy(rows_t, acc.at[de_t], add=True)

    plsc.subcore_barrier()
    pltpu.sync_copy(acc.at[pl.ds(s * 640, 640)],
                    p_out.at[c, pl.ds(s * 640, 640)])


_agg = pl.kernel(
    _agg_body,
    out_type=jax.ShapeDtypeStruct((NC, NPAD, D), jnp.float32),
    mesh=_mesh,
    scratch_types=[
        pltpu.VMEM((CH,), jnp.int32),
        pltpu.VMEM((CH,), jnp.int32),
        pltpu.VMEM((CH, D), jnp.float32),
        pltpu.VMEM((CH,), jnp.int32),
        pltpu.VMEM((CH,), jnp.int32),
        pltpu.VMEM((CH, D), jnp.float32),
        pltpu.VMEM((TAIL,), jnp.int32),
        pltpu.VMEM((TAIL,), jnp.int32),
        pltpu.VMEM((TAIL, D), jnp.float32),
        pltpu.VMEM_SHARED((NPAD, D), jnp.float32),
        pltpu.SemaphoreType.DMA,
        pltpu.SemaphoreType.DMA,
    ],
)


def _k2_body(g_ref, degO_ref, f_out, fs_out):
    g = g_ref[...]
    nrm = jnp.sqrt(jnp.sum(g * g, axis=1, keepdims=True))
    f = g / (nrm + 1e-12)
    dd = degO_ref[0] + degO_ref[1]
    ns = lax.rsqrt(jnp.maximum(dd, 1.0))
    f_out[...] = f
    fs_out[...] = f * ns


def _k2(g, degO3):
    return pl.pallas_call(
        _k2_body,
        grid=(NT,),
        in_specs=[
            pl.BlockSpec((RT, D), lambda i: (i, 0)),
            pl.BlockSpec((NC, RT, 1), lambda i: (0, i, 0)),
        ],
        out_specs=[
            pl.BlockSpec((RT, D), lambda i: (i, 0)),
            pl.BlockSpec((RT, D), lambda i: (i, 0)),
        ],
        out_shape=[
            jax.ShapeDtypeStruct((N, D), jnp.float32),
            jax.ShapeDtypeStruct((N, D), jnp.float32),
        ],
    )(g, degO3)


def _k4_body(p_ref, degI_ref, degO_ref, f_ref,
             wxr_ref, bxr_ref, wxz_ref, bxz_ref, wxh_ref, bxh_ref,
             whr_ref, bhr_ref, whz_ref, bhz_ref,
             rhs_out, z_out, xh_out):
    nd = lax.rsqrt(jnp.maximum(degI_ref[0] + degI_ref[1], 1.0))
    agg = (p_ref[0] + p_ref[1]) * nd
    wr = wxr_ref[...] + whr_ref[...]
    wz = wxz_ref[...] + whz_ref[...]
    br = bxr_ref[...] + bhr_ref[...]
    bz = bxz_ref[...] + bhz_ref[...]
    r = jax.nn.sigmoid(jnp.dot(agg, wr, preferred_element_type=jnp.float32) + br)
    z = jax.nn.sigmoid(jnp.dot(agg, wz, preferred_element_type=jnp.float32) + bz)
    xh = jnp.dot(agg, wxh_ref[...], preferred_element_type=jnp.float32) + bxh_ref[...]
    ns = lax.rsqrt(jnp.maximum(degO_ref[0] + degO_ref[1], 1.0))
    rhs_out[...] = r * f_ref[...] * ns
    z_out[...] = z
    xh_out[...] = xh


def _k4(p, degI3, degO3, feat, wxr, bxr, wxz, bxz, wxh, bxh, whr, bhr, whz, bhz):
    wspec = pl.BlockSpec((D, D), lambda i: (0, 0))
    bspec = pl.BlockSpec((1, D), lambda i: (0, 0))
    dspec = pl.BlockSpec((NC, RT, 1), lambda i: (0, i, 0))
    nspec = pl.BlockSpec((RT, D), lambda i: (i, 0))
    return pl.pallas_call(
        _k4_body,
        grid=(NT,),
        in_specs=[
            pl.BlockSpec((NC, RT, D), lambda i: (0, i, 0)),
            dspec, dspec, nspec,
            wspec, bspec, wspec, bspec, wspec, bspec,
            wspec, bspec, wspec, bspec,
        ],
        out_specs=[nspec, nspec, nspec],
        out_shape=[
            jax.ShapeDtypeStruct((N, D), jnp.float32),
            jax.ShapeDtypeStruct((N, D), jnp.float32),
            jax.ShapeDtypeStruct((N, D), jnp.float32),
        ],
    )(p, degI3, degO3, feat, wxr, bxr, wxz, bxz, wxh, bxh, whr, bhr, whz, bhz)


def _k6_body(q_ref, degI_ref, z_ref, xh_ref, f_ref, whh_ref, bhh_ref,
             tmax_ref, wu_ref, f2n_out, fu_out):
    nd = lax.rsqrt(jnp.maximum(degI_ref[0] + degI_ref[1], 1.0))
    agg2 = (q_ref[0] + q_ref[1]) * nd
    u = jnp.tanh(xh_ref[...]
                 + jnp.dot(agg2, whh_ref[...], preferred_element_type=jnp.float32)
                 + bhh_ref[...])
    f = f_ref[...]
    dh = (1.0 - z_ref[...]) * (u - f)
    dt = jnp.max(tmax_ref[...]) * 0.1
    f2 = f + dt * dh
    f2n = f2 / jnp.sqrt(jnp.sum(f2 * f2, axis=1, keepdims=True))
    f2n_out[...] = f2n
    fu_out[...] = jnp.dot(f2n, wu_ref[...], preferred_element_type=jnp.float32)


def _k6(q, degI3, z, xh, feat, whh, bhh, tmax, wu):
    nspec = pl.BlockSpec((RT, D), lambda i: (i, 0))
    return pl.pallas_call(
        _k6_body,
        grid=(NT,),
        in_specs=[
            pl.BlockSpec((NC, RT, D), lambda i: (0, i, 0)),
            pl.BlockSpec((NC, RT, 1), lambda i: (0, i, 0)),
            nspec, nspec, nspec,
            pl.BlockSpec((D, D), lambda i: (0, 0)),
            pl.BlockSpec((1, D), lambda i: (0, 0)),
            pl.BlockSpec((NW, 16), lambda i: (0, 0)),
            pl.BlockSpec((D, D), lambda i: (0, 0)),
        ],
        out_specs=[nspec, nspec],
        out_shape=[
            jax.ShapeDtypeStruct((N, D), jnp.float32),
            jax.ShapeDtypeStruct((N, D), jnp.float32),
        ],
    )(q, degI3, z, xh, feat, whh, bhh, tmax, wu)


def _attn_body(f3_ref, fu3_ref, wv_ref, bv_ref, we_ref, wsr_ref, sr_out):
    f3 = f3_ref[...]
    fu3 = fu3_ref[...]
    last = f3[:, SEG - 1, :]
    fv = jnp.dot(last, wv_ref[...], preferred_element_type=jnp.float32) \
        + bv_ref[...]
    sig = jax.nn.sigmoid(fu3 + fv[:, None, :])
    e3 = jnp.sum(sig * we_ref[...], axis=2, keepdims=True)
    emax = jnp.max(e3, axis=1, keepdims=True)
    ee = jnp.exp(e3 - emax)
    es = jnp.sum(ee, axis=1, keepdims=True)
    alpha = ee / es
    sr_g = jnp.sum(f3 * alpha, axis=1)
    srh = jnp.concatenate([last, sr_g], axis=1)
    sr0 = jnp.dot(srh, wsr_ref[...], preferred_element_type=jnp.float32)
    nrm = jnp.sqrt(jnp.sum(sr0 * sr0, axis=1, keepdims=True))
    sr_out[...] = sr0 / (nrm + 1e-12)


def _attn(f3, fu3, wv, bv, we, wsr):
    return pl.pallas_call(
        _attn_body,
        out_shape=jax.ShapeDtypeStruct((B, D), jnp.float32),
    )(f3, fu3, wv, bv, we, wsr)


def _k7_body(sr_ref, emb_ref, out_ref):
    et = emb_ref[...]
    sr = sr_ref[...]
    ones = jnp.ones((1, D), jnp.float32)
    rn2 = lax.dot_general(ones, et * et, (((1,), (1,)), ((), ())),
                          preferred_element_type=jnp.float32)
    logits = lax.dot_general(sr, et, (((1,), (1,)), ((), ())),
                             preferred_element_type=jnp.float32)
    out_ref[...] = logits * (SCALE / (jnp.sqrt(rn2) + 1e-12))


def _k7(sr, emb):
    return pl.pallas_call(
        _k7_body,
        grid=(NVT,),
        in_specs=[
            pl.BlockSpec((B, D), lambda i: (0, 0)),
            pl.BlockSpec((VT, D), lambda i: (i, 0)),
        ],
        out_specs=pl.BlockSpec((B, VT), lambda i: (0, i)),
        out_shape=jax.ShapeDtypeStruct((B, V), jnp.float32),
    )(sr, emb)


def kernel(iid, edge_index, edge_t, segment_ids, last_nodes, emb,
           Wxr, bxr, Wxz, bxz, Wxh, bxh, Whr, bhr, Whz, bhz, Whh, bhh,
           Wu, Wv, bv, We, Wsr):
    iid = iid.astype(jnp.int32)
    src = edge_index[0].astype(jnp.int32)
    dst = edge_index[1].astype(jnp.int32)
    edge_t = edge_t.astype(jnp.float32)
    z1 = jnp.zeros((NPAD,), jnp.float32)
    z2 = jnp.zeros((NPAD, D), jnp.float32)

    g, degO, degI, de, tmax = _k1(iid, src, dst, edge_t, emb, z1)
    degO3 = degO.reshape(NC, NPAD, 1)
    degI3 = degI.reshape(NC, NPAD, 1)

    feat, fs = _k2(g, degO3)
    p = _agg(fs, src, de, z2)
    rhs, z, xh = _k4(p, degI3, degO3, feat,
                     Wxr, bxr.reshape(1, D), Wxz, bxz.reshape(1, D),
                     Wxh, bxh.reshape(1, D), Whr, bhr.reshape(1, D),
                     Whz, bhz.reshape(1, D))
    q = _agg(rhs, src, de, z2)
    f2n, fu = _k6(q, degI3, z, xh, feat, Whh, bhh.reshape(1, D), tmax, Wu)

    f3 = f2n.reshape(B, SEG, D)
    fu3 = fu.reshape(B, SEG, D)
    sr = _attn(f3, fu3, Wv, bv.reshape(1, D), We.reshape(1, 1, D), Wsr)

    return _k7(sr, emb)


# speedup vs baseline: 9.8106x; 1.1515x over previous
import jax
import jax.numpy as jnp
from jax import lax
from jax.experimental import pallas as pl
from jax.experimental.pallas import tpu as pltpu
from jax.experimental.pallas import tpu_sc as plsc

N = 10000
E = 320000
D = 128
V = 100000
B = 100
SEG = N // B
SCALE = 12.0

NC = 2
NS = 16
NW = NC * NS
NPAD = 10240
DUMP = N
EPT = E // NW
CH = 80
NCHUNK = EPT // CH
GPT = 320
GCH = 64
NT = 10
RT = N // NT
VT = 2048
NVT = (V + VT - 1) // VT

_mesh = plsc.VectorSubcoreMesh(
    core_axis_name="c", subcore_axis_name="s", num_cores=NC, num_subcores=NS)


def _k1_body(iid_hbm, src_hbm, dst_hbm, et_hbm, emb_hbm, z1_hbm,
             g_out, degO_out, degI_out, ide_out, tmax_out,
             idx_v, rows_v, src_all, dst_all, et_all,
             sde0, sde1, dsti0, dsti1, m_v, tm_v,
             degO_acc, degI_acc, gsem, wsem0, wsem1):
    c = lax.axis_index("c")
    s = lax.axis_index("s")
    w = c * NS + s

    @pl.when(s == 0)
    def _():
        pltpu.sync_copy(z1_hbm, degO_acc)
        pltpu.sync_copy(z1_hbm, degI_acc)

    start = jnp.minimum(w * GPT, N - GPT)
    pltpu.sync_copy(iid_hbm.at[pl.ds(start, GPT)], idx_v)
    for j in range(GPT // GCH):
        pltpu.async_copy(emb_hbm.at[idx_v.at[pl.ds(j * GCH, GCH)]],
                         rows_v.at[pl.ds(j * GCH, GCH)], gsem)

    base = w * EPT
    pltpu.sync_copy(src_hbm.at[pl.ds(base, EPT)], src_all)
    pltpu.sync_copy(dst_hbm.at[pl.ds(base, EPT)], dst_all)
    pltpu.sync_copy(et_hbm.at[pl.ds(base, EPT)], et_all)

    plsc.subcore_barrier()

    tm_v[...] = jnp.full((16,), -3.0e38, jnp.float32)

    def _build(i, sde, dsti):
        for j in range(CH // 16):
            sl = pl.ds(i * CH + j * 16, 16)
            t = et_all[sl]
            d = dst_all[sl]
            msk = t <= 0.0
            m_v[pl.ds(j * 16, 16)] = jnp.where(msk, 1.0, 0.0)
            sde[0, pl.ds(j * 16, 16)] = src_all[sl]
            sde[1, pl.ds(j * 16, 16)] = jnp.where(msk, d, DUMP + (d & 127))
            dsti[pl.ds(j * 16, 16)] = d
            tm_v[...] = jnp.maximum(tm_v[...], t)

    def _scatter(sde, dsti):
        pltpu.sync_copy(m_v, degO_acc.at[sde.at[0]], add=True)
        pltpu.sync_copy(m_v, degI_acc.at[dsti], add=True)

    def group(g, _):
        for b, sde, dsti, wsem in ((0, sde0, dsti0, wsem0),
                                   (1, sde1, dsti1, wsem1)):
            i = 2 * g + b

            @pl.when(g > 0)
            def _():
                pltpu.make_async_copy(sde, ide_out.at[w, 0], wsem).wait()

            _build(i, sde, dsti)
            _scatter(sde, dsti)
            pltpu.async_copy(sde, ide_out.at[w, i], wsem)
        return 0

    lax.fori_loop(0, NCHUNK // 2, group, 0)

    pltpu.make_async_copy(sde0, ide_out.at[w, 0], wsem0).wait()
    _build(NCHUNK - 1, sde0, dsti0)
    _scatter(sde0, dsti0)
    pltpu.sync_copy(sde0, ide_out.at[w, NCHUNK - 1])
    pltpu.make_async_copy(sde1, ide_out.at[w, 0], wsem1).wait()

    pltpu.sync_copy(tm_v, tmax_out.at[w])

    for j in range(GPT // GCH):
        pltpu.make_async_copy(emb_hbm.at[idx_v.at[pl.ds(j * GCH, GCH)]],
                              rows_v.at[pl.ds(j * GCH, GCH)], gsem).wait()
    pltpu.sync_copy(rows_v, g_out.at[pl.ds(start, GPT)])

    plsc.subcore_barrier()
    pltpu.sync_copy(degO_acc.at[pl.ds(s * 640, 640)],
                    degO_out.at[c, pl.ds(s * 640, 640)])
    pltpu.sync_copy(degI_acc.at[pl.ds(s * 640, 640)],
                    degI_out.at[c, pl.ds(s * 640, 640)])


NCHT = NCHUNK

_k1 = pl.kernel(
    _k1_body,
    out_type=(
        jax.ShapeDtypeStruct((N, D), jnp.float32),
        jax.ShapeDtypeStruct((NC, NPAD), jnp.float32),
        jax.ShapeDtypeStruct((NC, NPAD), jnp.float32),
        jax.ShapeDtypeStruct((NW, NCHT, 2, CH), jnp.int32),
        jax.ShapeDtypeStruct((NW, 16), jnp.float32),
    ),
    mesh=_mesh,
    scratch_types=[
        pltpu.VMEM((GPT,), jnp.int32),
        pltpu.VMEM((GPT, D), jnp.float32),
        pltpu.VMEM((EPT,), jnp.int32),
        pltpu.VMEM((EPT,), jnp.int32),
        pltpu.VMEM((EPT,), jnp.float32),
        pltpu.VMEM((2, CH), jnp.int32),
        pltpu.VMEM((2, CH), jnp.int32),
        pltpu.VMEM((CH,), jnp.int32),
        pltpu.VMEM((CH,), jnp.int32),
        pltpu.VMEM((CH,), jnp.float32),
        pltpu.VMEM((16,), jnp.float32),
        pltpu.VMEM_SHARED((NPAD,), jnp.float32),
        pltpu.VMEM_SHARED((NPAD,), jnp.float32),
        pltpu.SemaphoreType.DMA,
        pltpu.SemaphoreType.DMA,
        pltpu.SemaphoreType.DMA,
    ],
)


NBUF = 4


def _agg_body(x_hbm, ide_hbm, z2_hbm, p_out,
              i0, i1, i2, i3, r0, r1, r2, r3, acc,
              g0, g1, g2, g3, s0, s1, s2, s3):
    c = lax.axis_index("c")
    s = lax.axis_index("s")
    w = c * NS + s
    idx = (i0, i1, i2, i3)
    rows = (r0, r1, r2, r3)
    gs = (g0, g1, g2, g3)
    ss = (s0, s1, s2, s3)

    pltpu.sync_copy(z2_hbm.at[pl.ds(s * 640, 640)], acc.at[pl.ds(s * 640, 640)])
    plsc.subcore_barrier()

    def _wait_scatter(b):
        pltpu.make_async_copy(x_hbm.at[pl.ds(0, CH)], rows[b], ss[b]).wait()

    def _wait_gather(b):
        pltpu.make_async_copy(x_hbm.at[pl.ds(0, CH)], rows[b], gs[b]).wait()

    def group(g, _):
        for b in range(NBUF):
            i = NBUF * g + b

            @pl.when(g > 0)
            def _(b=b):
                _wait_scatter(b)

            pltpu.sync_copy(ide_hbm.at[w, i], idx[b])
            pltpu.async_copy(x_hbm.at[idx[b].at[0]], rows[b], gs[b])
        for b in range(NBUF):
            _wait_gather(b)
            pltpu.async_copy(rows[b], acc.at[idx[b].at[1]], ss[b], add=True)
        return 0

    lax.fori_loop(0, NCHUNK // NBUF, group, 0)

    _wait_scatter(0)
    pltpu.sync_copy(ide_hbm.at[w, NCHUNK - 1], idx[0])
    pltpu.async_copy(x_hbm.at[idx[0].at[0]], rows[0], gs[0]).wait()
    pltpu.async_copy(rows[0], acc.at[idx[0].at[1]], ss[0], add=True)

    for b in range(NBUF):
        _wait_scatter(b)

    plsc.subcore_barrier()
    pltpu.sync_copy(acc.at[pl.ds(s * 640, 640)],
                    p_out.at[c, pl.ds(s * 640, 640)])


_agg = pl.kernel(
    _agg_body,
    out_type=jax.ShapeDtypeStruct((NC, NPAD, D), jnp.float32),
    mesh=_mesh,
    scratch_types=(
        [pltpu.VMEM((2, CH), jnp.int32) for _ in range(NBUF)]
        + [pltpu.VMEM((CH, D), jnp.float32) for _ in range(NBUF)]
        + [pltpu.VMEM_SHARED((NPAD, D), jnp.float32)]
        + [pltpu.SemaphoreType.DMA] * (2 * NBUF)
    ),
)


def _k2_body(g_ref, degO_ref, f_out, fs_out):
    g = g_ref[...]
    nrm = jnp.sqrt(jnp.sum(g * g, axis=1, keepdims=True))
    f = g / (nrm + 1e-12)
    dd = degO_ref[0] + degO_ref[1]
    ns = lax.rsqrt(jnp.maximum(dd, 1.0))
    f_out[...] = f
    fs_out[...] = f * ns


def _k2(g, degO3):
    return pl.pallas_call(
        _k2_body,
        grid=(NT,),
        in_specs=[
            pl.BlockSpec((RT, D), lambda i: (i, 0)),
            pl.BlockSpec((NC, RT, 1), lambda i: (0, i, 0)),
        ],
        out_specs=[
            pl.BlockSpec((RT, D), lambda i: (i, 0)),
            pl.BlockSpec((RT, D), lambda i: (i, 0)),
        ],
        out_shape=[
            jax.ShapeDtypeStruct((N, D), jnp.float32),
            jax.ShapeDtypeStruct((N, D), jnp.float32),
        ],
    )(g, degO3)


def _k4_body(p_ref, degI_ref, degO_ref, f_ref,
             wxr_ref, bxr_ref, wxz_ref, bxz_ref, wxh_ref, bxh_ref,
             whr_ref, bhr_ref, whz_ref, bhz_ref,
             rhs_out, z_out, xh_out):
    nd = lax.rsqrt(jnp.maximum(degI_ref[0] + degI_ref[1], 1.0))
    agg = (p_ref[0] + p_ref[1]) * nd
    wr = wxr_ref[...] + whr_ref[...]
    wz = wxz_ref[...] + whz_ref[...]
    br = bxr_ref[...] + bhr_ref[...]
    bz = bxz_ref[...] + bhz_ref[...]
    r = jax.nn.sigmoid(jnp.dot(agg, wr, preferred_element_type=jnp.float32) + br)
    z = jax.nn.sigmoid(jnp.dot(agg, wz, preferred_element_type=jnp.float32) + bz)
    xh = jnp.dot(agg, wxh_ref[...], preferred_element_type=jnp.float32) + bxh_ref[...]
    ns = lax.rsqrt(jnp.maximum(degO_ref[0] + degO_ref[1], 1.0))
    rhs_out[...] = r * f_ref[...] * ns
    z_out[...] = z
    xh_out[...] = xh


def _k4(p, degI3, degO3, feat, wxr, bxr, wxz, bxz, wxh, bxh, whr, bhr, whz, bhz):
    wspec = pl.BlockSpec((D, D), lambda i: (0, 0))
    bspec = pl.BlockSpec((1, D), lambda i: (0, 0))
    dspec = pl.BlockSpec((NC, RT, 1), lambda i: (0, i, 0))
    nspec = pl.BlockSpec((RT, D), lambda i: (i, 0))
    return pl.pallas_call(
        _k4_body,
        grid=(NT,),
        in_specs=[
            pl.BlockSpec((NC, RT, D), lambda i: (0, i, 0)),
            dspec, dspec, nspec,
            wspec, bspec, wspec, bspec, wspec, bspec,
            wspec, bspec, wspec, bspec,
        ],
        out_specs=[nspec, nspec, nspec],
        out_shape=[
            jax.ShapeDtypeStruct((N, D), jnp.float32),
            jax.ShapeDtypeStruct((N, D), jnp.float32),
            jax.ShapeDtypeStruct((N, D), jnp.float32),
        ],
    )(p, degI3, degO3, feat, wxr, bxr, wxz, bxz, wxh, bxh, whr, bhr, whz, bhz)


def _k6_body(q_ref, degI_ref, z_ref, xh_ref, f_ref, whh_ref, bhh_ref,
             tmax_ref, wu_ref, f2n_out, fu_out):
    nd = lax.rsqrt(jnp.maximum(degI_ref[0] + degI_ref[1], 1.0))
    agg2 = (q_ref[0] + q_ref[1]) * nd
    u = jnp.tanh(xh_ref[...]
                 + jnp.dot(agg2, whh_ref[...], preferred_element_type=jnp.float32)
                 + bhh_ref[...])
    f = f_ref[...]
    dh = (1.0 - z_ref[...]) * (u - f)
    dt = jnp.max(tmax_ref[...]) * 0.1
    f2 = f + dt * dh
    f2n = f2 / jnp.sqrt(jnp.sum(f2 * f2, axis=1, keepdims=True))
    f2n_out[...] = f2n
    fu_out[...] = jnp.dot(f2n, wu_ref[...], preferred_element_type=jnp.float32)


def _k6(q, degI3, z, xh, feat, whh, bhh, tmax, wu):
    nspec = pl.BlockSpec((RT, D), lambda i: (i, 0))
    return pl.pallas_call(
        _k6_body,
        grid=(NT,),
        in_specs=[
            pl.BlockSpec((NC, RT, D), lambda i: (0, i, 0)),
            pl.BlockSpec((NC, RT, 1), lambda i: (0, i, 0)),
            nspec, nspec, nspec,
            pl.BlockSpec((D, D), lambda i: (0, 0)),
            pl.BlockSpec((1, D), lambda i: (0, 0)),
            pl.BlockSpec((NW, 16), lambda i: (0, 0)),
            pl.BlockSpec((D, D), lambda i: (0, 0)),
        ],
        out_specs=[nspec, nspec],
        out_shape=[
            jax.ShapeDtypeStruct((N, D), jnp.float32),
            jax.ShapeDtypeStruct((N, D), jnp.float32),
        ],
    )(q, degI3, z, xh, feat, whh, bhh, tmax, wu)


def _attn_body(f3_ref, fu3_ref, wv_ref, bv_ref, we_ref, wsr_ref, sr_out):
    f3 = f3_ref[...]
    fu3 = fu3_ref[...]
    last = f3[:, SEG - 1, :]
    fv = jnp.dot(last, wv_ref[...], preferred_element_type=jnp.float32) \
        + bv_ref[...]
    sig = jax.nn.sigmoid(fu3 + fv[:, None, :])
    e3 = jnp.sum(sig * we_ref[...], axis=2, keepdims=True)
    emax = jnp.max(e3, axis=1, keepdims=True)
    ee = jnp.exp(e3 - emax)
    es = jnp.sum(ee, axis=1, keepdims=True)
    alpha = ee / es
    sr_g = jnp.sum(f3 * alpha, axis=1)
    srh = jnp.concatenate([last, sr_g], axis=1)
    sr0 = jnp.dot(srh, wsr_ref[...], preferred_element_type=jnp.float32)
    nrm = jnp.sqrt(jnp.sum(sr0 * sr0, axis=1, keepdims=True))
    sr_out[...] = sr0 / (nrm + 1e-12)


def _attn(f3, fu3, wv, bv, we, wsr):
    return pl.pallas_call(
        _attn_body,
        out_shape=jax.ShapeDtypeStruct((B, D), jnp.float32),
    )(f3, fu3, wv, bv, we, wsr)


def _k7_body(sr_ref, emb_ref, out_ref):
    et = emb_ref[...]
    sr = sr_ref[...]
    ones = jnp.ones((1, D), jnp.float32)
    rn2 = lax.dot_general(ones, et * et, (((1,), (1,)), ((), ())),
                          preferred_element_type=jnp.float32)
    logits = lax.dot_general(sr, et, (((1,), (1,)), ((), ())),
                             preferred_element_type=jnp.float32)
    out_ref[...] = logits * (SCALE / (jnp.sqrt(rn2) + 1e-12))


def _k7(sr, emb):
    return pl.pallas_call(
        _k7_body,
        grid=(NVT,),
        in_specs=[
            pl.BlockSpec((B, D), lambda i: (0, 0)),
            pl.BlockSpec((VT, D), lambda i: (i, 0)),
        ],
        out_specs=pl.BlockSpec((B, VT), lambda i: (0, i)),
        out_shape=jax.ShapeDtypeStruct((B, V), jnp.float32),
    )(sr, emb)


def kernel(iid, edge_index, edge_t, segment_ids, last_nodes, emb,
           Wxr, bxr, Wxz, bxz, Wxh, bxh, Whr, bhr, Whz, bhz, Whh, bhh,
           Wu, Wv, bv, We, Wsr):
    iid = iid.astype(jnp.int32)
    src = edge_index[0].astype(jnp.int32)
    dst = edge_index[1].astype(jnp.int32)
    edge_t = edge_t.astype(jnp.float32)
    z1 = jnp.zeros((NPAD,), jnp.float32)
    z2 = jnp.zeros((NPAD, D), jnp.float32)

    g, degO, degI, ide, tmax = _k1(iid, src, dst, edge_t, emb, z1)
    degO3 = degO.reshape(NC, NPAD, 1)
    degI3 = degI.reshape(NC, NPAD, 1)

    feat, fs = _k2(g, degO3)
    p = _agg(fs, ide, z2)
    rhs, z, xh = _k4(p, degI3, degO3, feat,
                     Wxr, bxr.reshape(1, D), Wxz, bxz.reshape(1, D),
                     Wxh, bxh.reshape(1, D), Whr, bhr.reshape(1, D),
                     Whz, bhz.reshape(1, D))
    q = _agg(rhs, ide, z2)
    f2n, fu = _k6(q, degI3, z, xh, feat, Whh, bhh.reshape(1, D), tmax, Wu)

    f3 = f2n.reshape(B, SEG, D)
    fu3 = fu.reshape(B, SEG, D)
    sr = _attn(f3, fu3, Wv, bv.reshape(1, D), We.reshape(1, 1, D), Wsr)

    return _k7(sr, emb)


# speedup vs baseline: 9.8350x; 1.0025x over previous
import jax
import jax.numpy as jnp
from jax import lax
from jax.experimental import pallas as pl
from jax.experimental.pallas import tpu as pltpu
from jax.experimental.pallas import tpu_sc as plsc

N = 10000
E = 320000
D = 128
V = 100000
B = 100
SEG = N // B
SCALE = 12.0

NC = 2
NS = 16
NW = NC * NS
NPAD = 10240
DUMP = N
EPT = E // NW
CH = 80
NCHUNK = EPT // CH
GPT = 320
GCH = 64
NT = 10
RT = N // NT
VT = 2048
NVT = (V + VT - 1) // VT

_mesh = plsc.VectorSubcoreMesh(
    core_axis_name="c", subcore_axis_name="s", num_cores=NC, num_subcores=NS)


def _k1_body(iid_hbm, src_hbm, dst_hbm, et_hbm, emb_hbm, z1_hbm,
             g_out, degO_out, degI_out, ide_out, tmax_out,
             idx_v, rows_v, src_all, dst_all, et_all,
             sde0, sde1, dsti0, dsti1, m_v, tm_v,
             degO_acc, degI_acc, gsem, wsem0, wsem1):
    c = lax.axis_index("c")
    s = lax.axis_index("s")
    w = c * NS + s

    @pl.when(s == 0)
    def _():
        pltpu.sync_copy(z1_hbm, degO_acc)
        pltpu.sync_copy(z1_hbm, degI_acc)

    start = jnp.minimum(w * GPT, N - GPT)
    pltpu.sync_copy(iid_hbm.at[pl.ds(start, GPT)], idx_v)
    for j in range(GPT // GCH):
        pltpu.async_copy(emb_hbm.at[idx_v.at[pl.ds(j * GCH, GCH)]],
                         rows_v.at[pl.ds(j * GCH, GCH)], gsem)

    base = w * EPT
    pltpu.sync_copy(src_hbm.at[pl.ds(base, EPT)], src_all)
    pltpu.sync_copy(dst_hbm.at[pl.ds(base, EPT)], dst_all)
    pltpu.sync_copy(et_hbm.at[pl.ds(base, EPT)], et_all)

    plsc.subcore_barrier()

    tm_v[...] = jnp.full((16,), -3.0e38, jnp.float32)

    def _build(i, sde, dsti):
        for j in range(CH // 16):
            sl = pl.ds(i * CH + j * 16, 16)
            t = et_all[sl]
            d = dst_all[sl]
            msk = t <= 0.0
            m_v[pl.ds(j * 16, 16)] = jnp.where(msk, 1.0, 0.0)
            sde[0, pl.ds(j * 16, 16)] = src_all[sl]
            sde[1, pl.ds(j * 16, 16)] = jnp.where(msk, d, DUMP + (d & 127))
            dsti[pl.ds(j * 16, 16)] = d
            tm_v[...] = jnp.maximum(tm_v[...], t)

    def _scatter(sde, dsti):
        pltpu.sync_copy(m_v, degO_acc.at[sde.at[0]], add=True)
        pltpu.sync_copy(m_v, degI_acc.at[dsti], add=True)

    def group(g, _):
        for b, sde, dsti, wsem in ((0, sde0, dsti0, wsem0),
                                   (1, sde1, dsti1, wsem1)):
            i = 2 * g + b

            @pl.when(g > 0)
            def _():
                pltpu.make_async_copy(sde, ide_out.at[w, 0], wsem).wait()

            _build(i, sde, dsti)
            _scatter(sde, dsti)
            pltpu.async_copy(sde, ide_out.at[w, i], wsem)
        return 0

    lax.fori_loop(0, NCHUNK // 2, group, 0)

    pltpu.make_async_copy(sde0, ide_out.at[w, 0], wsem0).wait()
    _build(NCHUNK - 1, sde0, dsti0)
    _scatter(sde0, dsti0)
    pltpu.sync_copy(sde0, ide_out.at[w, NCHUNK - 1])
    pltpu.make_async_copy(sde1, ide_out.at[w, 0], wsem1).wait()

    pltpu.sync_copy(tm_v, tmax_out.at[w])

    for j in range(GPT // GCH):
        pltpu.make_async_copy(emb_hbm.at[idx_v.at[pl.ds(j * GCH, GCH)]],
                              rows_v.at[pl.ds(j * GCH, GCH)], gsem).wait()
    pltpu.sync_copy(rows_v, g_out.at[pl.ds(start, GPT)])

    plsc.subcore_barrier()
    pltpu.sync_copy(degO_acc.at[pl.ds(s * 640, 640)],
                    degO_out.at[c, pl.ds(s * 640, 640)])
    pltpu.sync_copy(degI_acc.at[pl.ds(s * 640, 640)],
                    degI_out.at[c, pl.ds(s * 640, 640)])


NCHT = NCHUNK

_k1 = pl.kernel(
    _k1_body,
    out_type=(
        jax.ShapeDtypeStruct((N, D), jnp.float32),
        jax.ShapeDtypeStruct((NC, NPAD), jnp.float32),
        jax.ShapeDtypeStruct((NC, NPAD), jnp.float32),
        jax.ShapeDtypeStruct((NW, NCHT, 2, CH), jnp.int32),
        jax.ShapeDtypeStruct((NW, 16), jnp.float32),
    ),
    mesh=_mesh,
    scratch_types=[
        pltpu.VMEM((GPT,), jnp.int32),
        pltpu.VMEM((GPT, D), jnp.float32),
        pltpu.VMEM((EPT,), jnp.int32),
        pltpu.VMEM((EPT,), jnp.int32),
        pltpu.VMEM((EPT,), jnp.float32),
        pltpu.VMEM((2, CH), jnp.int32),
        pltpu.VMEM((2, CH), jnp.int32),
        pltpu.VMEM((CH,), jnp.int32),
        pltpu.VMEM((CH,), jnp.int32),
        pltpu.VMEM((CH,), jnp.float32),
        pltpu.VMEM((16,), jnp.float32),
        pltpu.VMEM_SHARED((NPAD,), jnp.float32),
        pltpu.VMEM_SHARED((NPAD,), jnp.float32),
        pltpu.SemaphoreType.DMA,
        pltpu.SemaphoreType.DMA,
        pltpu.SemaphoreType.DMA,
    ],
)


NBUF = 4


def _agg_body(x_hbm, ide_hbm, z2_hbm, p_out,
              i0, i1, i2, i3, r0, r1, r2, r3, acc,
              g0, g1, g2, g3, s0, s1, s2, s3):
    c = lax.axis_index("c")
    s = lax.axis_index("s")
    w = c * NS + s
    idx = (i0, i1, i2, i3)
    rows = (r0, r1, r2, r3)
    gs = (g0, g1, g2, g3)
    ss = (s0, s1, s2, s3)

    pltpu.sync_copy(z2_hbm.at[pl.ds(s * 640, 640)], acc.at[pl.ds(s * 640, 640)])
    plsc.subcore_barrier()

    def _wait_scatter(b):
        pltpu.make_async_copy(x_hbm.at[pl.ds(0, CH)], rows[b], ss[b]).wait()

    def _wait_gather(b):
        pltpu.make_async_copy(x_hbm.at[pl.ds(0, CH)], rows[b], gs[b]).wait()

    def group(g, _):
        for b in range(NBUF):
            i = NBUF * g + b

            @pl.when(g > 0)
            def _(b=b):
                _wait_scatter(b)

            pltpu.sync_copy(ide_hbm.at[w, i], idx[b])
            pltpu.async_copy(x_hbm.at[idx[b].at[0]], rows[b], gs[b])
        for b in range(NBUF):
            _wait_gather(b)
            pltpu.async_copy(rows[b], acc.at[idx[b].at[1]], ss[b], add=True)
        return 0

    lax.fori_loop(0, NCHUNK // NBUF, group, 0)

    _wait_scatter(0)
    pltpu.sync_copy(ide_hbm.at[w, NCHUNK - 1], idx[0])
    pltpu.async_copy(x_hbm.at[idx[0].at[0]], rows[0], gs[0]).wait()
    pltpu.async_copy(rows[0], acc.at[idx[0].at[1]], ss[0], add=True)

    for b in range(NBUF):
        _wait_scatter(b)

    plsc.subcore_barrier()
    pltpu.sync_copy(acc.at[pl.ds(s * 640, 640)],
                    p_out.at[c, pl.ds(s * 640, 640)])


_agg = pl.kernel(
    _agg_body,
    out_type=jax.ShapeDtypeStruct((NC, NPAD, D), jnp.float32),
    mesh=_mesh,
    scratch_types=(
        [pltpu.VMEM((2, CH), jnp.int32) for _ in range(NBUF)]
        + [pltpu.VMEM((CH, D), jnp.float32) for _ in range(NBUF)]
        + [pltpu.VMEM_SHARED((NPAD, D), jnp.float32)]
        + [pltpu.SemaphoreType.DMA] * (2 * NBUF)
    ),
)


def _k2_body(g_ref, degO_ref, f_out, fs_out):
    g = g_ref[...]
    nrm = jnp.sqrt(jnp.sum(g * g, axis=1, keepdims=True))
    f = g / (nrm + 1e-12)
    dd = degO_ref[0] + degO_ref[1]
    ns = lax.rsqrt(jnp.maximum(dd, 1.0))
    f_out[...] = f
    fs_out[...] = f * ns


def _k2(g, degO3):
    return pl.pallas_call(
        _k2_body,
        grid=(NT,),
        in_specs=[
            pl.BlockSpec((RT, D), lambda i: (i, 0)),
            pl.BlockSpec((NC, RT, 1), lambda i: (0, i, 0)),
        ],
        out_specs=[
            pl.BlockSpec((RT, D), lambda i: (i, 0)),
            pl.BlockSpec((RT, D), lambda i: (i, 0)),
        ],
        out_shape=[
            jax.ShapeDtypeStruct((N, D), jnp.float32),
            jax.ShapeDtypeStruct((N, D), jnp.float32),
        ],
    )(g, degO3)


def _k4_body(p_ref, degI_ref, degO_ref, f_ref,
             wxr_ref, bxr_ref, wxz_ref, bxz_ref, wxh_ref, bxh_ref,
             whr_ref, bhr_ref, whz_ref, bhz_ref,
             rhs_out, z_out, xh_out):
    nd = lax.rsqrt(jnp.maximum(degI_ref[0] + degI_ref[1], 1.0))
    agg = (p_ref[0] + p_ref[1]) * nd
    wr = wxr_ref[...] + whr_ref[...]
    wz = wxz_ref[...] + whz_ref[...]
    br = bxr_ref[...] + bhr_ref[...]
    bz = bxz_ref[...] + bhz_ref[...]
    r = jax.nn.sigmoid(jnp.dot(agg, wr, preferred_element_type=jnp.float32) + br)
    z = jax.nn.sigmoid(jnp.dot(agg, wz, preferred_element_type=jnp.float32) + bz)
    xh = jnp.dot(agg, wxh_ref[...], preferred_element_type=jnp.float32) + bxh_ref[...]
    ns = lax.rsqrt(jnp.maximum(degO_ref[0] + degO_ref[1], 1.0))
    rhs_out[...] = r * f_ref[...] * ns
    z_out[...] = z
    xh_out[...] = xh


def _k4(p, degI3, degO3, feat, wxr, bxr, wxz, bxz, wxh, bxh, whr, bhr, whz, bhz):
    wspec = pl.BlockSpec((D, D), lambda i: (0, 0))
    bspec = pl.BlockSpec((1, D), lambda i: (0, 0))
    dspec = pl.BlockSpec((NC, RT, 1), lambda i: (0, i, 0))
    nspec = pl.BlockSpec((RT, D), lambda i: (i, 0))
    return pl.pallas_call(
        _k4_body,
        grid=(NT,),
        in_specs=[
            pl.BlockSpec((NC, RT, D), lambda i: (0, i, 0)),
            dspec, dspec, nspec,
            wspec, bspec, wspec, bspec, wspec, bspec,
            wspec, bspec, wspec, bspec,
        ],
        out_specs=[nspec, nspec, nspec],
        out_shape=[
            jax.ShapeDtypeStruct((N, D), jnp.float32),
            jax.ShapeDtypeStruct((N, D), jnp.float32),
            jax.ShapeDtypeStruct((N, D), jnp.float32),
        ],
    )(p, degI3, degO3, feat, wxr, bxr, wxz, bxz, wxh, bxh, whr, bhr, whz, bhz)


def _k6_body(q_ref, degI_ref, z_ref, xh_ref, f_ref, whh_ref, bhh_ref,
             tmax_ref, wu_ref, f2n_out, fu_out):
    nd = lax.rsqrt(jnp.maximum(degI_ref[0] + degI_ref[1], 1.0))
    agg2 = (q_ref[0] + q_ref[1]) * nd
    u = jnp.tanh(xh_ref[...]
                 + jnp.dot(agg2, whh_ref[...], preferred_element_type=jnp.float32)
                 + bhh_ref[...])
    f = f_ref[...]
    dh = (1.0 - z_ref[...]) * (u - f)
    dt = jnp.max(tmax_ref[...]) * 0.1
    f2 = f + dt * dh
    f2n = f2 / jnp.sqrt(jnp.sum(f2 * f2, axis=1, keepdims=True))
    f2n_out[...] = f2n
    fu_out[...] = jnp.dot(f2n, wu_ref[...], preferred_element_type=jnp.float32)


def _k6(q, degI3, z, xh, feat, whh, bhh, tmax, wu):
    nspec = pl.BlockSpec((RT, D), lambda i: (i, 0))
    return pl.pallas_call(
        _k6_body,
        grid=(NT,),
        in_specs=[
            pl.BlockSpec((NC, RT, D), lambda i: (0, i, 0)),
            pl.BlockSpec((NC, RT, 1), lambda i: (0, i, 0)),
            nspec, nspec, nspec,
            pl.BlockSpec((D, D), lambda i: (0, 0)),
            pl.BlockSpec((1, D), lambda i: (0, 0)),
            pl.BlockSpec((NW, 16), lambda i: (0, 0)),
            pl.BlockSpec((D, D), lambda i: (0, 0)),
        ],
        out_specs=[nspec, nspec],
        out_shape=[
            jax.ShapeDtypeStruct((N, D), jnp.float32),
            jax.ShapeDtypeStruct((N, D), jnp.float32),
        ],
    )(q, degI3, z, xh, feat, whh, bhh, tmax, wu)


def _k7_body(f3_ref, fu3_ref, wv_ref, bv_ref, we_ref, wsr_ref, emb_ref,
             out_ref, sr_s):
    @pl.when(pl.program_id(0) == 0)
    def _():
        f3 = f3_ref[...]
        fu3 = fu3_ref[...]
        last = f3[:, SEG - 1, :]
        fv = jnp.dot(last, wv_ref[...], preferred_element_type=jnp.float32) \
            + bv_ref[...]
        sig = jax.nn.sigmoid(fu3 + fv[:, None, :])
        e3 = jnp.sum(sig * we_ref[...], axis=2, keepdims=True)
        emax = jnp.max(e3, axis=1, keepdims=True)
        ee = jnp.exp(e3 - emax)
        es = jnp.sum(ee, axis=1, keepdims=True)
        alpha = ee / es
        sr_g = jnp.sum(f3 * alpha, axis=1)
        srh = jnp.concatenate([last, sr_g], axis=1)
        sr0 = jnp.dot(srh, wsr_ref[...], preferred_element_type=jnp.float32)
        nrm = jnp.sqrt(jnp.sum(sr0 * sr0, axis=1, keepdims=True))
        sr_s[...] = sr0 / (nrm + 1e-12)

    et = emb_ref[...]
    sr = sr_s[...]
    ones = jnp.ones((1, D), jnp.float32)
    rn2 = lax.dot_general(ones, et * et, (((1,), (1,)), ((), ())),
                          preferred_element_type=jnp.float32)
    logits = lax.dot_general(sr, et, (((1,), (1,)), ((), ())),
                             preferred_element_type=jnp.float32)
    out_ref[...] = logits * (SCALE / (jnp.sqrt(rn2) + 1e-12))


def _k7(f3, fu3, wv, bv, we, wsr, emb):
    cspec = lambda shape: pl.BlockSpec(shape, lambda i: (0,) * len(shape))
    return pl.pallas_call(
        _k7_body,
        grid=(NVT,),
        in_specs=[
            cspec((B, SEG, D)),
            cspec((B, SEG, D)),
            cspec((D, D)),
            cspec((1, D)),
            cspec((1, 1, D)),
            cspec((2 * D, D)),
            pl.BlockSpec((VT, D), lambda i: (i, 0)),
        ],
        out_specs=pl.BlockSpec((B, VT), lambda i: (0, i)),
        out_shape=jax.ShapeDtypeStruct((B, V), jnp.float32),
        scratch_shapes=[pltpu.VMEM((B, D), jnp.float32)],
    )(f3, fu3, wv, bv, we, wsr, emb)


def kernel(iid, edge_index, edge_t, segment_ids, last_nodes, emb,
           Wxr, bxr, Wxz, bxz, Wxh, bxh, Whr, bhr, Whz, bhz, Whh, bhh,
           Wu, Wv, bv, We, Wsr):
    iid = iid.astype(jnp.int32)
    src = edge_index[0].astype(jnp.int32)
    dst = edge_index[1].astype(jnp.int32)
    edge_t = edge_t.astype(jnp.float32)
    z1 = jnp.zeros((NPAD,), jnp.float32)
    z2 = jnp.zeros((NPAD, D), jnp.float32)

    g, degO, degI, ide, tmax = _k1(iid, src, dst, edge_t, emb, z1)
    degO3 = degO.reshape(NC, NPAD, 1)
    degI3 = degI.reshape(NC, NPAD, 1)

    feat, fs = _k2(g, degO3)
    p = _agg(fs, ide, z2)
    rhs, z, xh = _k4(p, degI3, degO3, feat,
                     Wxr, bxr.reshape(1, D), Wxz, bxz.reshape(1, D),
                     Wxh, bxh.reshape(1, D), Whr, bhr.reshape(1, D),
                     Whz, bhz.reshape(1, D))
    q = _agg(rhs, ide, z2)
    f2n, fu = _k6(q, degI3, z, xh, feat, Whh, bhh.reshape(1, D), tmax, Wu)

    f3 = f2n.reshape(B, SEG, D)
    fu3 = fu.reshape(B, SEG, D)
    return _k7(f3, fu3, Wv, bv.reshape(1, D), We.reshape(1, 1, D), Wsr, emb)
